# baseline (speedup 1.0000x reference)
import functools

import jax
import jax.numpy as jnp
from jax.experimental import pallas as pl
from jax.experimental.pallas import tpu as pltpu

_PACK = (8, 4, 2, 2)


def _round8(n):
    return (n + 7) & ~7


def _stage(v, H, p, x3a_ref, x3b_ref, wa_ref, ba_ref, wb_ref, bb_ref,
           sc_ref, sh_ref):
    Hp = H + 2
    U = H // p
    Upt = x3a_ref.shape[1]
    cmid_p = x3b_ref.shape[2] // 3
    cout_g = sc_ref.shape[1]
    cout_p = 2 * cout_g

    def im2col(x3_ref, v, c):
        @pl.when(pl.program_id(0) == 0)
        def _zero_halo():
            x3_ref[0:1] = jnp.zeros((1, Upt, 3 * c), jnp.float32)
            x3_ref[Hp - 1:Hp] = jnp.zeros((1, Upt, 3 * c), jnp.float32)
            x3_ref[:, 0:1, :] = jnp.zeros((Hp, 1, 3 * c), jnp.float32)
            x3_ref[:, U - 1:U, :] = jnp.zeros((Hp, 1, 3 * c), jnp.float32)

        x3_ref[1:H + 1, 1:U + 1, 0:c] = v
        x3_ref[1:H + 1, 0:U, c:2 * c] = v
        x3_ref[1:H + 1, 0:U - 1, 2 * c:3 * c] = v[:, 1:U, :]

    def conv(x3_ref, w_ref, b_ref, co):
        flat = x3_ref[...].reshape(Hp * Upt, x3_ref.shape[2])
        q = jnp.dot(flat, w_ref[...], preferred_element_type=jnp.float32)
        q = q.reshape(Hp, Upt, 3 * co)
        acc = q[0:H, 0:U, 0:co]
        for dy in (1, 2):
            acc = acc + q[dy:dy + H, 0:U, dy * co:(dy + 1) * co]
        return jnp.maximum(acc + b_ref[...].reshape(1, 1, co), 0.0)

    im2col(x3a_ref, v, v.shape[2])
    y = conv(x3a_ref, wa_ref, ba_ref, cmid_p)
    im2col(x3b_ref, y, cmid_p)
    y2 = conv(x3b_ref, wb_ref, bb_ref, cout_p)

    y2r = y2.reshape(H // 2, 2, U, cout_p)
    zh = jnp.maximum(y2r[:, 0], y2r[:, 1])
    c1 = cout_p // p
    parts = []
    for k in range(p // 2):
        parts.append(jnp.maximum(zh[:, :, (2 * k) * c1:(2 * k + 1) * c1],
                                 zh[:, :, (2 * k + 1) * c1:(2 * k + 2) * c1]))
    z = parts[0] if len(parts) == 1 else jnp.concatenate(parts, axis=-1)
    return z * sc_ref[...].reshape(1, 1, cout_g) + \
        sh_ref[...].reshape(1, 1, cout_g)


def _trunk_kernel(H0, packs, *refs):
    x_ref = refs[0]
    o_ref = refs[25]
    scratches = refs[26:]
    v = x_ref[0]
    H = H0
    for i, p in enumerate(packs):
        params = refs[1 + 6 * i:7 + 6 * i]
        z = _stage(v, H, p, scratches[2 * i], scratches[2 * i + 1], *params)
        H //= 2
        if i + 1 < len(packs):
            pn = packs[i + 1]
            if pn == p // 2:
                v = z
            else:
                pairs = [jnp.concatenate([z[:, 2 * u2:2 * u2 + 1, :],
                                          z[:, 2 * u2 + 1:2 * u2 + 2, :]],
                                         axis=-1)
                         for u2 in range(z.shape[1] // 2)]
                v = jnp.concatenate(pairs, axis=1)
    o_ref[0] = z


def _const_spec(a):
    nd = a.ndim
    return pl.BlockSpec(a.shape, lambda i, _nd=nd: (0,) * _nd)


def _run_trunk(xq, flat_params, H0):
    n = xq.shape[0]
    u0, cin_p = xq.shape[2], xq.shape[3]

    in_specs = [pl.BlockSpec((1, H0, u0, cin_p), lambda i: (i, 0, 0, 0))]
    in_specs += [_const_spec(a) for a in flat_params]

    scratch = []
    H = H0
    c = cin_p // _PACK[0]
    for i, p in enumerate(_PACK):
        wa2 = flat_params[6 * i]
        cmid_p = wa2.shape[1] // 3
        hp, upt = H + 2, _round8(H // p)
        scratch.append(pltpu.VMEM((hp, upt, 3 * p * c), jnp.float32))
        scratch.append(pltpu.VMEM((hp, upt, 3 * cmid_p), jnp.float32))
        c = flat_params[6 * i + 2].shape[1] // 6
        c = c // (p // 2)
        H //= 2

    hf = H0 // 16
    kern = functools.partial(_trunk_kernel, H0, _PACK)
    return pl.pallas_call(
        kern,
        out_shape=jax.ShapeDtypeStruct((n, hf, hf, c), jnp.float32),
        grid=(n,),
        in_specs=in_specs,
        out_specs=pl.BlockSpec((1, hf, hf, c), lambda i: (i, 0, 0, 0)),
        scratch_shapes=scratch,
        compiler_params=pltpu.CompilerParams(
            dimension_semantics=("arbitrary",),
            vmem_limit_bytes=100 * 1024 * 1024,
        ),
    )(xq, *flat_params)


def _classifier_kernel(x_ref, w1_ref, b1_ref, w2_ref, b2_ref, o_ref):
    h = jnp.dot(x_ref[...], w1_ref[...], preferred_element_type=jnp.float32)
    h = jnp.maximum(h + b1_ref[...], 0.0)
    o_ref[...] = jnp.dot(h, w2_ref[...],
                         preferred_element_type=jnp.float32) + b2_ref[...]


def _pack_w(wk, p):
    cin, cout = wk.shape[1], wk.shape[2]
    w = jnp.zeros((p * cin, 9, p * cout), jnp.float32)
    for dy in range(3):
        for b in range(3):
            t = dy * 3 + b
            for jin in range(p):
                for jout in range(p):
                    dx = p * (b - 1) + jin - jout
                    if -1 <= dx <= 1:
                        w = w.at[jin * cin:(jin + 1) * cin, t,
                                 jout * cout:(jout + 1) * cout].set(
                                     wk[dy * 3 + dx + 1])
    w = w.reshape(p * cin, 3, 3, p * cout).transpose(2, 0, 1, 3)
    return w.reshape(3 * p * cin, 3 * p * cout)


def kernel(x, s0_wa, s0_ba, s0_wb, s0_bb, s0_sc, s0_sh,
           s1_wa, s1_ba, s1_wb, s1_bb, s1_sc, s1_sh,
           s2_wa, s2_ba, s2_wb, s2_bb, s2_sc, s2_sh,
           s3_wa, s3_ba, s3_wb, s3_bb, s3_sc, s3_sh,
           fc1_w, fc1_b, fc2_w, fc2_b):
    n = x.shape[0]
    stages = [
        (s0_wa, s0_ba, s0_wb, s0_bb, s0_sc, s0_sh),
        (s1_wa, s1_ba, s1_wb, s1_bb, s1_sc, s1_sh),
        (s2_wa, s2_ba, s2_wb, s2_bb, s2_sc, s2_sh),
        (s3_wa, s3_ba, s3_wb, s3_bb, s3_sc, s3_sh),
    ]

    xt = jnp.transpose(x, (0, 2, 3, 1)).astype(jnp.float32)
    H0 = xt.shape[1]
    p0 = _PACK[0]
    xq = xt.reshape(n, H0, H0 // p0, p0 * xt.shape[3])

    flat_params = []
    for (wa, ba, wb, bb, sc, sh), p in zip(stages, _PACK):
        flat_params += [_pack_w(wa, p), jnp.tile(ba, (1, p)),
                        _pack_w(wb, p), jnp.tile(bb, (1, p)),
                        jnp.tile(sc, (1, p // 2)), jnp.tile(sh, (1, p // 2))]

    cur = _run_trunk(xq, flat_params, H0)
    feats = cur.reshape(n, -1)
    out_c = fc2_w.shape[1]
    return pl.pallas_call(
        _classifier_kernel,
        out_shape=jax.ShapeDtypeStruct((n, out_c), jnp.float32),
    )(feats, fc1_w, fc1_b, fc2_w, fc2_b)


# speedup vs baseline: 1.0246x; 1.0246x over previous
import functools

import jax
import jax.numpy as jnp
from jax.experimental import pallas as pl
from jax.experimental.pallas import tpu as pltpu

_PACK = (8, 4, 2, 2)


def _round8(n):
    return (n + 7) & ~7


def _stage(v, H, p, x3a_ref, x3b_ref, wa_ref, ba_ref, wb_ref, bb_ref,
           sc_ref, sh_ref):
    Hp = H + 2
    U = H // p
    Upt = x3a_ref.shape[1]
    cmid_p = x3b_ref.shape[2] // 3
    cout_g = sc_ref.shape[1]
    cout_p = 2 * cout_g

    def im2col(x3_ref, v, c):
        x3_ref[0:1] = jnp.zeros((1, Upt, 3 * c), jnp.float32)
        x3_ref[Hp - 1:Hp] = jnp.zeros((1, Upt, 3 * c), jnp.float32)
        x3_ref[:, 0:1, :] = jnp.zeros((Hp, 1, 3 * c), jnp.float32)
        x3_ref[:, U - 1:U, :] = jnp.zeros((Hp, 1, 3 * c), jnp.float32)
        x3_ref[1:H + 1, 1:U + 1, 0:c] = v
        x3_ref[1:H + 1, 0:U, c:2 * c] = v
        x3_ref[1:H + 1, 0:U - 1, 2 * c:3 * c] = v[:, 1:U, :]

    def conv(x3_ref, w_ref, b_ref, co):
        flat = x3_ref[...].reshape(Hp * Upt, x3_ref.shape[2])
        q = jnp.dot(flat, w_ref[...], preferred_element_type=jnp.float32)
        q = q.reshape(Hp, Upt, 3 * co)
        acc = q[0:H, 0:U, 0:co]
        for dy in (1, 2):
            acc = acc + q[dy:dy + H, 0:U, dy * co:(dy + 1) * co]
        return jnp.maximum(acc + b_ref[...].reshape(1, 1, co), 0.0)

    im2col(x3a_ref, v, v.shape[2])
    y = conv(x3a_ref, wa_ref, ba_ref, cmid_p)
    im2col(x3b_ref, y, cmid_p)
    y2 = conv(x3b_ref, wb_ref, bb_ref, cout_p)

    y2r = y2.reshape(H // 2, 2, U, cout_p)
    zh = jnp.maximum(y2r[:, 0], y2r[:, 1])
    c1 = cout_p // p
    parts = []
    for k in range(p // 2):
        parts.append(jnp.maximum(zh[:, :, (2 * k) * c1:(2 * k + 1) * c1],
                                 zh[:, :, (2 * k + 1) * c1:(2 * k + 2) * c1]))
    z = parts[0] if len(parts) == 1 else jnp.concatenate(parts, axis=-1)
    return z * sc_ref[...].reshape(1, 1, cout_g) + \
        sh_ref[...].reshape(1, 1, cout_g)


def _trunk_kernel(H0, packs, *refs):
    x_ref = refs[0]
    o_ref = refs[25]
    scratches = refs[26:]
    v = x_ref[0]
    H = H0
    for i, p in enumerate(packs):
        params = refs[1 + 6 * i:7 + 6 * i]
        z = _stage(v, H, p, scratches[2 * i], scratches[2 * i + 1], *params)
        H //= 2
        if i + 1 < len(packs):
            pn = packs[i + 1]
            if pn == p // 2:
                v = z
            else:
                pairs = [jnp.concatenate([z[:, 2 * u2:2 * u2 + 1, :],
                                          z[:, 2 * u2 + 1:2 * u2 + 2, :]],
                                         axis=-1)
                         for u2 in range(z.shape[1] // 2)]
                v = jnp.concatenate(pairs, axis=1)
    o_ref[0] = z


def _const_spec(a):
    nd = a.ndim
    return pl.BlockSpec(a.shape, lambda i, _nd=nd: (0,) * _nd)


def _run_trunk(xq, flat_params, H0):
    n = xq.shape[0]
    u0, cin_p = xq.shape[2], xq.shape[3]

    in_specs = [pl.BlockSpec((1, H0, u0, cin_p), lambda i: (i, 0, 0, 0))]
    in_specs += [_const_spec(a) for a in flat_params]

    scratch = []
    H = H0
    c = cin_p // _PACK[0]
    for i, p in enumerate(_PACK):
        wa2 = flat_params[6 * i]
        cmid_p = wa2.shape[1] // 3
        hp, upt = H + 2, _round8(H // p)
        scratch.append(pltpu.VMEM((hp, upt, 3 * p * c), jnp.float32))
        scratch.append(pltpu.VMEM((hp, upt, 3 * cmid_p), jnp.float32))
        c = flat_params[6 * i + 2].shape[1] // 6
        c = c // (p // 2)
        H //= 2

    hf = H0 // 16
    kern = functools.partial(_trunk_kernel, H0, _PACK)
    return pl.pallas_call(
        kern,
        out_shape=jax.ShapeDtypeStruct((n, hf, hf, c), jnp.float32),
        grid=(n,),
        in_specs=in_specs,
        out_specs=pl.BlockSpec((1, hf, hf, c), lambda i: (i, 0, 0, 0)),
        scratch_shapes=scratch,
        compiler_params=pltpu.CompilerParams(
            dimension_semantics=("parallel",),
            vmem_limit_bytes=100 * 1024 * 1024,
        ),
    )(xq, *flat_params)


def _classifier_kernel(x_ref, w1_ref, b1_ref, w2_ref, b2_ref, o_ref):
    h = jnp.dot(x_ref[...], w1_ref[...], preferred_element_type=jnp.float32)
    h = jnp.maximum(h + b1_ref[...], 0.0)
    o_ref[...] = jnp.dot(h, w2_ref[...],
                         preferred_element_type=jnp.float32) + b2_ref[...]


def _pack_w(wk, p):
    cin, cout = wk.shape[1], wk.shape[2]
    w = jnp.zeros((p * cin, 9, p * cout), jnp.float32)
    for dy in range(3):
        for b in range(3):
            t = dy * 3 + b
            for jin in range(p):
                for jout in range(p):
                    dx = p * (b - 1) + jin - jout
                    if -1 <= dx <= 1:
                        w = w.at[jin * cin:(jin + 1) * cin, t,
                                 jout * cout:(jout + 1) * cout].set(
                                     wk[dy * 3 + dx + 1])
    w = w.reshape(p * cin, 3, 3, p * cout).transpose(2, 0, 1, 3)
    return w.reshape(3 * p * cin, 3 * p * cout)


def kernel(x, s0_wa, s0_ba, s0_wb, s0_bb, s0_sc, s0_sh,
           s1_wa, s1_ba, s1_wb, s1_bb, s1_sc, s1_sh,
           s2_wa, s2_ba, s2_wb, s2_bb, s2_sc, s2_sh,
           s3_wa, s3_ba, s3_wb, s3_bb, s3_sc, s3_sh,
           fc1_w, fc1_b, fc2_w, fc2_b):
    n = x.shape[0]
    stages = [
        (s0_wa, s0_ba, s0_wb, s0_bb, s0_sc, s0_sh),
        (s1_wa, s1_ba, s1_wb, s1_bb, s1_sc, s1_sh),
        (s2_wa, s2_ba, s2_wb, s2_bb, s2_sc, s2_sh),
        (s3_wa, s3_ba, s3_wb, s3_bb, s3_sc, s3_sh),
    ]

    xt = jnp.transpose(x, (0, 2, 3, 1)).astype(jnp.float32)
    H0 = xt.shape[1]
    p0 = _PACK[0]
    xq = xt.reshape(n, H0, H0 // p0, p0 * xt.shape[3])

    flat_params = []
    for (wa, ba, wb, bb, sc, sh), p in zip(stages, _PACK):
        flat_params += [_pack_w(wa, p), jnp.tile(ba, (1, p)),
                        _pack_w(wb, p), jnp.tile(bb, (1, p)),
                        jnp.tile(sc, (1, p // 2)), jnp.tile(sh, (1, p // 2))]

    cur = _run_trunk(xq, flat_params, H0)
    feats = cur.reshape(n, -1)
    out_c = fc2_w.shape[1]
    return pl.pallas_call(
        _classifier_kernel,
        out_shape=jax.ShapeDtypeStruct((n, out_c), jnp.float32),
    )(feats, fc1_w, fc1_b, fc2_w, fc2_b)
